# baseline (speedup 1.0000x reference)
import jax
import jax.numpy as jnp
from jax import lax
from jax.experimental import pallas as pl
from jax.experimental.pallas import tpu as pltpu
from jax.experimental.pallas import tpu_sc as plsc

N = 10000
E = 160000
DIN, DHID, DOUT = 256, 512, 128
NC, NS = 2, 16
NPAD = 10240
NH = NPAD // 2
TRASH = 64
NACC = NH + TRASH
K = 80
NB = (E // NS) // K
KL = 128
NBL = 88
NPT = NPAD // NS
HPT = NPAD // (NC * NS)
RPT = NH // NS
WBR = 80
BN = 1024

_MESH = plsc.VectorSubcoreMesh(
    core_axis_name="c", subcore_axis_name="s", num_cores=NC, num_subcores=NS)


def _part_body(src3_h, dst3_h, ones_h, deg_h, lists_h, cnt_h,
               src_v, dst_v, ones_v, dv_v, la_s, la_d, lb_s, lb_d, cnt_v,
               deg_sh):
  cid = lax.axis_index("c")
  sid = lax.axis_index("s")
  pltpu.sync_copy(src3_h.at[sid], src_v)
  pltpu.sync_copy(dst3_h.at[sid], dst_v)

  @pl.when(cid == 1)
  def _():
    pltpu.sync_copy(ones_h.at[pl.ds(0, NPT)], ones_v)
    pltpu.sync_copy(ones_v, deg_sh.at[pl.ds(sid * NPT, NPT)])

  plsc.subcore_barrier()

  @pl.when(cid == 1)
  def _():
    def body(nb, _):
      pltpu.sync_copy(ones_v.at[pl.ds(0, K)], deg_sh.at[dst_v.at[nb]],
                      add=True)
      return ()

    lax.fori_loop(0, NB, body, ())

  @pl.when(cid == 0)
  def _():
    i16 = lax.iota(jnp.int32, 16)

    def grp(b, c):
      ca, cb = c
      for j in range(K // 16):
        sv = src_v[b, pl.ds(j * 16, 16)]
        d = dst_v[b, pl.ds(j * 16, 16)]
        m = d < NH
        mi = m.astype(jnp.int32)
        pa = ca + plsc.cumsum(mi) - 1
        pb = cb + plsc.cumsum(1 - mi) - 1
        for sref, dref, pos, msk, dt in (
            (la_s, la_d, pa, m, d), (lb_s, lb_d, pb, ~m, d - NH)):
          r = pos >> 7
          q = pos & (KL - 1)
          plsc.store_scatter(sref, [r, q], sv, mask=msk)
          plsc.store_scatter(dref, [r, q], dt, mask=msk)
        ca = ca + jnp.sum(mi)
        cb = cb + (16 - jnp.sum(mi))
      return (ca, cb)

    ca, cb = lax.fori_loop(0, NB, grp, (jnp.int32(0), jnp.int32(0)))
    for sref, dref, cur in ((la_s, la_d, ca), (lb_s, lb_d, cb)):
      for j in range(2 * KL // 16):
        pos = cur + j * 16 + i16
        r = pos >> 7
        q = pos & (KL - 1)
        plsc.store_scatter(sref, [r, q], jnp.zeros((16,), jnp.int32))
        plsc.store_scatter(dref, [r, q], NH + (pos & (TRASH - 1)))
    nba = ((ca + KL - 1) // KL + 1) & ~1
    nbb = ((cb + KL - 1) // KL + 1) & ~1
    cnt_v[...] = jnp.where(i16 == 0, nba, jnp.where(i16 == 1, nbb, 0))
    pltpu.sync_copy(cnt_v, cnt_h.at[pl.ds(sid * 16, 16)])
    pltpu.sync_copy(la_s, lists_h.at[sid, 0])
    pltpu.sync_copy(la_d, lists_h.at[sid, 1])
    pltpu.sync_copy(lb_s, lists_h.at[sid, 2])
    pltpu.sync_copy(lb_d, lists_h.at[sid, 3])

  plsc.subcore_barrier()

  @pl.when(cid == 1)
  def _():
    pltpu.sync_copy(deg_sh.at[pl.ds(sid * NPT, NPT)], dv_v)
    pltpu.sync_copy(dv_v, deg_h.at[pl.ds(sid * NPT, NPT)])


_part = pl.kernel(
    _part_body,
    out_type=(
        jax.ShapeDtypeStruct((NPAD,), jnp.float32),
        jax.ShapeDtypeStruct((NS, 4, NBL, KL), jnp.int32),
        jax.ShapeDtypeStruct((NS * 16,), jnp.int32),
    ),
    mesh=_MESH,
    scratch_types=[
        pltpu.VMEM((NB, K), jnp.int32),
        pltpu.VMEM((NB, K), jnp.int32),
        pltpu.VMEM((NPT,), jnp.float32),
        pltpu.VMEM((NPT,), jnp.float32),
        pltpu.VMEM((NBL, KL), jnp.int32),
        pltpu.VMEM((NBL, KL), jnp.int32),
        pltpu.VMEM((NBL, KL), jnp.int32),
        pltpu.VMEM((NBL, KL), jnp.int32),
        pltpu.VMEM((16,), jnp.int32),
        pltpu.VMEM_SHARED((NPAD,), jnp.float32),
    ],
    compiler_params=pltpu.CompilerParams(needs_layout_passes=False),
)


def _edge_loop(tbl, src_v, dst_v, rows_v, acc_sh, sem0, nb):
  @pl.when(nb > 0)
  def _():
    pltpu.async_copy(tbl.at[src_v.at[0]], rows_v.at[0], sem0)

    def body(b, _):
      buf = lax.rem(b, 2)

      @pl.when(b + 1 < nb)
      def _():
        pltpu.async_copy(tbl.at[src_v.at[b + 1]], rows_v.at[1 - buf], sem0)

      pltpu.make_async_copy(tbl.at[src_v.at[b]], rows_v.at[buf], sem0).wait()
      pltpu.sync_copy(rows_v.at[buf], acc_sh.at[dst_v.at[b]], add=True)
      return ()

    lax.fori_loop(0, nb, body, ())


def _one_pass(tbl, out2d, base, bucket, sid, lists_h, cnt_v, src_v, dst_v,
              rows_v, wb_v, acc_sh, sem0):
  pltpu.sync_copy(lists_h.at[sid, 2 * bucket], src_v)
  pltpu.sync_copy(lists_h.at[sid, 2 * bucket + 1], dst_v)
  i16 = lax.iota(jnp.int32, 16)
  nb = jnp.sum(jnp.where(i16 == bucket, cnt_v[...], 0))
  for j in range(RPT // WBR):
    r = pl.multiple_of(base + sid * RPT + j * WBR, 8)
    a = pl.multiple_of(sid * RPT + j * WBR, 8)
    pltpu.sync_copy(tbl.at[pl.ds(r, WBR)], wb_v)
    pltpu.sync_copy(wb_v, acc_sh.at[pl.ds(a, WBR)])
  plsc.subcore_barrier()
  _edge_loop(tbl, src_v, dst_v, rows_v, acc_sh, sem0, nb)
  plsc.subcore_barrier()
  for j in range(RPT // WBR):
    r = pl.multiple_of(base + sid * RPT + j * WBR, 8)
    a = pl.multiple_of(sid * RPT + j * WBR, 8)
    pltpu.sync_copy(acc_sh.at[pl.ds(a, WBR)], wb_v)
    pltpu.sync_copy(wb_v, out2d.at[pl.ds(r, WBR)])
  plsc.subcore_barrier()


_PROP_SCRATCH = [
    pltpu.VMEM((NBL, KL), jnp.int32),
    pltpu.VMEM((NBL, KL), jnp.int32),
    pltpu.VMEM((2, KL, 128), jnp.float32),
    pltpu.VMEM((WBR, 128), jnp.float32),
    pltpu.VMEM((16,), jnp.int32),
    pltpu.VMEM_SHARED((NACC, 128), jnp.float32),
    pltpu.SemaphoreType.DMA,
]


def _prop1_body(table_h, lists_h, cnt_h, out_h, src_v, dst_v, rows_v,
                wb_v, cnt_v, acc_sh, sem0):
  cid = lax.axis_index("c")
  sid = lax.axis_index("s")
  pltpu.sync_copy(cnt_h.at[pl.ds(sid * 16, 16)], cnt_v)
  for p in range(2):
    _one_pass(table_h.at[cid], out_h.at[cid], p * NH, p, sid, lists_h,
              cnt_v, src_v, dst_v, rows_v, wb_v, acc_sh, sem0)


_prop1 = pl.kernel(
    _prop1_body,
    out_type=jax.ShapeDtypeStruct((NC, NPAD, DIN // 2), jnp.float32),
    mesh=_MESH,
    scratch_types=_PROP_SCRATCH,
    compiler_params=pltpu.CompilerParams(needs_layout_passes=False),
)


def _prop2_body(table_h, lists_h, cnt_h, out_h, src_v, dst_v, rows_v,
                wb_v, cnt_v, acc_sh, sem0):
  cid = lax.axis_index("c")
  sid = lax.axis_index("s")
  pltpu.sync_copy(cnt_h.at[pl.ds(sid * 16, 16)], cnt_v)
  base = pl.multiple_of(cid * NH, 8)
  _one_pass(table_h, out_h, base, cid, sid, lists_h, cnt_v, src_v, dst_v,
            rows_v, wb_v, acc_sh, sem0)


_prop2 = pl.kernel(
    _prop2_body,
    out_type=jax.ShapeDtypeStruct((NPAD, DOUT), jnp.float32),
    mesh=_MESH,
    scratch_types=_PROP_SCRATCH,
    compiler_params=pltpu.CompilerParams(needs_layout_passes=False),
)


def _scale_body(x_ref, deg_ref, out_ref, dinv_ref):
  dinv = lax.rsqrt(deg_ref[...])
  dinv_ref[...] = dinv
  y = x_ref[...] * dinv
  out_ref[0] = y[:, : DIN // 2]
  out_ref[1] = y[:, DIN // 2 :]


_scale_split = pl.pallas_call(
    _scale_body,
    grid=(NPAD // BN,),
    in_specs=[
        pl.BlockSpec((BN, DIN), lambda i: (i, 0)),
        pl.BlockSpec((BN, 1), lambda i: (i, 0)),
    ],
    out_specs=[
        pl.BlockSpec((NC, BN, DIN // 2), lambda i: (0, i, 0)),
        pl.BlockSpec((BN, 1), lambda i: (i, 0)),
    ],
    out_shape=[
        jax.ShapeDtypeStruct((NC, NPAD, DIN // 2), jnp.float32),
        jax.ShapeDtypeStruct((NPAD, 1), jnp.float32),
    ],
)


def _mid_body(s1_ref, dinv_ref, w1_ref, b1_ref, w2_ref, out_ref):
  s = jnp.concatenate([s1_ref[0], s1_ref[1]], axis=1)
  p = s * dinv_ref[...]
  h = jnp.dot(p, w1_ref[...], preferred_element_type=jnp.float32)
  h = jnp.maximum(h + b1_ref[...][None, :], 0.0)
  y2 = jnp.dot(h, w2_ref[...], preferred_element_type=jnp.float32)
  out_ref[...] = y2 * dinv_ref[...]


_mid = pl.pallas_call(
    _mid_body,
    grid=(NPAD // BN,),
    in_specs=[
        pl.BlockSpec((NC, BN, DIN // 2), lambda i: (0, i, 0)),
        pl.BlockSpec((BN, 1), lambda i: (i, 0)),
        pl.BlockSpec((DIN, DHID), lambda i: (0, 0)),
        pl.BlockSpec((DHID,), lambda i: (0,)),
        pl.BlockSpec((DHID, DOUT), lambda i: (0, 0)),
    ],
    out_specs=pl.BlockSpec((BN, DOUT), lambda i: (i, 0)),
    out_shape=jax.ShapeDtypeStruct((NPAD, DOUT), jnp.float32),
)


def _final_body(s2_ref, dinv_ref, b2_ref, out_ref):
  out_ref[...] = s2_ref[...] * dinv_ref[...] + b2_ref[...][None, :]


_final = pl.pallas_call(
    _final_body,
    grid=(NPAD // BN,),
    in_specs=[
        pl.BlockSpec((BN, DOUT), lambda i: (i, 0)),
        pl.BlockSpec((BN, 1), lambda i: (i, 0)),
        pl.BlockSpec((DOUT,), lambda i: (0,)),
    ],
    out_specs=pl.BlockSpec((BN, DOUT), lambda i: (i, 0)),
    out_shape=jax.ShapeDtypeStruct((N, DOUT), jnp.float32),
)


def kernel(x, edge_index, W1, b1, W2, b2):
  src3 = edge_index[0].reshape(NS, NB, K)
  dst3 = edge_index[1].reshape(NS, NB, K)
  ones = jnp.ones((NPAD,), jnp.float32)

  deg, lists, cnt = _part(src3, dst3, ones)
  deg2 = deg.reshape(NPAD, 1)

  y1, dinv2 = _scale_split(x, deg2)
  s1 = _prop1(y1, lists, cnt)
  y2 = _mid(s1, dinv2, W1, b1, W2)
  s2 = _prop2(y2, lists, cnt)
  return _final(s2, dinv2, b2)


# speedup vs baseline: 1.4100x; 1.4100x over previous
import jax
import jax.numpy as jnp
from jax import lax
from jax.experimental import pallas as pl
from jax.experimental.pallas import tpu as pltpu
from jax.experimental.pallas import tpu_sc as plsc

N = 10000
E = 160000
DIN, DHID, DOUT = 256, 512, 128
NC, NS = 2, 16
NPAD = 10240
NH = NPAD // 2
TRASH = 64
NACC = NH + TRASH
K = 80
NB = (E // NS) // K
NPT = NPAD // NS
HPT = NPAD // (NC * NS)
RPT = NH // NS
WBR = 80
BN = 1024

_MESH = plsc.VectorSubcoreMesh(
    core_axis_name="c", subcore_axis_name="s", num_cores=NC, num_subcores=NS)


def _deg_body(dst3_h, ones_h, deg_h, dst_v, ones_v, dv_v, deg_sh):
  cid = lax.axis_index("c")
  sid = lax.axis_index("s")
  pltpu.sync_copy(dst3_h.at[sid], dst_v)
  pltpu.sync_copy(ones_h.at[pl.ds(0, NPT)], ones_v)
  pltpu.sync_copy(ones_v, deg_sh.at[pl.ds(sid * NPT, NPT)])
  plsc.subcore_barrier()

  def body(nb, _):
    pltpu.sync_copy(ones_v.at[pl.ds(0, K)], deg_sh.at[dst_v.at[nb]], add=True)
    return ()

  lax.fori_loop(0, NB, body, ())
  plsc.subcore_barrier()

  row0 = (cid * NS + sid) * HPT
  pltpu.sync_copy(deg_sh.at[pl.ds(row0, HPT)], dv_v)
  pltpu.sync_copy(dv_v, deg_h.at[pl.ds(row0, HPT)])


_deg = pl.kernel(
    _deg_body,
    out_type=jax.ShapeDtypeStruct((NPAD,), jnp.float32),
    mesh=_MESH,
    scratch_types=[
        pltpu.VMEM((NB, K), jnp.int32),
        pltpu.VMEM((NPT,), jnp.float32),
        pltpu.VMEM((HPT,), jnp.float32),
        pltpu.VMEM_SHARED((NPAD,), jnp.float32),
    ],
)


def _transform_dst(dst_v, base):
  def row(b, _):
    for j in range(K // 16):
      d = dst_v[b, pl.ds(j * 16, 16)]
      idx = d - base
      ok = (idx >= 0) & (idx < NH)
      dst_v[b, pl.ds(j * 16, 16)] = jnp.where(
          ok, idx, NH + (d & (TRASH - 1)))
    return ()

  lax.fori_loop(0, NB, row, ())


def _edge_loop(tbl, src_v, dstt_v, rows_v, acc_sh, sem0, sem1):
  pltpu.async_copy(tbl.at[src_v.at[0]], rows_v.at[0], sem0)

  def body(t, _):
    b0 = 2 * t
    pltpu.async_copy(tbl.at[src_v.at[b0 + 1]], rows_v.at[1], sem1)
    pltpu.make_async_copy(tbl.at[src_v.at[b0]], rows_v.at[0], sem0).wait()
    pltpu.sync_copy(rows_v.at[0], acc_sh.at[dstt_v.at[b0]], add=True)
    pltpu.async_copy(tbl.at[src_v.at[b0 + 2]], rows_v.at[0], sem0)
    pltpu.make_async_copy(tbl.at[src_v.at[b0 + 1]], rows_v.at[1], sem1).wait()
    pltpu.sync_copy(rows_v.at[1], acc_sh.at[dstt_v.at[b0 + 1]], add=True)
    return ()

  lax.fori_loop(0, (NB - 1) // 2, body, ())
  last = NB - 1
  pltpu.make_async_copy(tbl.at[src_v.at[last]], rows_v.at[0], sem0).wait()
  pltpu.sync_copy(rows_v.at[0], acc_sh.at[dstt_v.at[last]], add=True)


def _one_pass(tbl, out2d, base, sid, dst3_h, src_v, dst_v, rows_v, wb_v,
              acc_sh, sem0, sem1):
  pltpu.sync_copy(dst3_h.at[sid], dst_v)
  _transform_dst(dst_v, base)
  for j in range(RPT // WBR):
    r = pl.multiple_of(base + sid * RPT + j * WBR, 8)
    a = pl.multiple_of(sid * RPT + j * WBR, 8)
    pltpu.sync_copy(tbl.at[pl.ds(r, WBR)], wb_v)
    pltpu.sync_copy(wb_v, acc_sh.at[pl.ds(a, WBR)])
  plsc.subcore_barrier()
  _edge_loop(tbl, src_v, dst_v, rows_v, acc_sh, sem0, sem1)
  plsc.subcore_barrier()
  for j in range(RPT // WBR):
    r = pl.multiple_of(base + sid * RPT + j * WBR, 8)
    a = pl.multiple_of(sid * RPT + j * WBR, 8)
    pltpu.sync_copy(acc_sh.at[pl.ds(a, WBR)], wb_v)
    pltpu.sync_copy(wb_v, out2d.at[pl.ds(r, WBR)])
  plsc.subcore_barrier()


_PROP_SCRATCH = [
    pltpu.VMEM((NB, K), jnp.int32),
    pltpu.VMEM((NB, K), jnp.int32),
    pltpu.VMEM((2, K, 128), jnp.float32),
    pltpu.VMEM((WBR, 128), jnp.float32),
    pltpu.VMEM_SHARED((NACC, 128), jnp.float32),
    pltpu.SemaphoreType.DMA,
    pltpu.SemaphoreType.DMA,
]


def _prop1_body(table_h, src3_h, dst3_h, out_h, src_v, dst_v, rows_v,
                wb_v, acc_sh, sem0, sem1):
  cid = lax.axis_index("c")
  sid = lax.axis_index("s")
  pltpu.sync_copy(src3_h.at[sid], src_v)
  for p in range(2):
    _one_pass(table_h.at[cid], out_h.at[cid], p * NH, sid, dst3_h, src_v,
              dst_v, rows_v, wb_v, acc_sh, sem0, sem1)


_prop1 = pl.kernel(
    _prop1_body,
    out_type=jax.ShapeDtypeStruct((NC, NPAD, DIN // 2), jnp.float32),
    mesh=_MESH,
    scratch_types=_PROP_SCRATCH,
)


def _prop2_body(table_h, src3_h, dst3_h, out_h, src_v, dst_v, rows_v,
                wb_v, acc_sh, sem0, sem1):
  cid = lax.axis_index("c")
  sid = lax.axis_index("s")
  pltpu.sync_copy(src3_h.at[sid], src_v)
  base = pl.multiple_of(cid * NH, 8)
  _one_pass(table_h, out_h, base, sid, dst3_h, src_v, dst_v, rows_v, wb_v,
            acc_sh, sem0, sem1)


_prop2 = pl.kernel(
    _prop2_body,
    out_type=jax.ShapeDtypeStruct((NPAD, DOUT), jnp.float32),
    mesh=_MESH,
    scratch_types=_PROP_SCRATCH,
)


def _scale_body(x_ref, deg_ref, out_ref, dinv_ref):
  dinv = lax.rsqrt(deg_ref[...])
  dinv_ref[...] = dinv
  y = x_ref[...] * dinv
  out_ref[0] = y[:, : DIN // 2]
  out_ref[1] = y[:, DIN // 2 :]


_scale_split = pl.pallas_call(
    _scale_body,
    grid=(NPAD // BN,),
    in_specs=[
        pl.BlockSpec((BN, DIN), lambda i: (i, 0)),
        pl.BlockSpec((BN, 1), lambda i: (i, 0)),
    ],
    out_specs=[
        pl.BlockSpec((NC, BN, DIN // 2), lambda i: (0, i, 0)),
        pl.BlockSpec((BN, 1), lambda i: (i, 0)),
    ],
    out_shape=[
        jax.ShapeDtypeStruct((NC, NPAD, DIN // 2), jnp.float32),
        jax.ShapeDtypeStruct((NPAD, 1), jnp.float32),
    ],
)


def _mid_body(s1_ref, dinv_ref, w1_ref, b1_ref, w2_ref, out_ref):
  s = jnp.concatenate([s1_ref[0], s1_ref[1]], axis=1)
  p = s * dinv_ref[...]
  h = jnp.dot(p, w1_ref[...], preferred_element_type=jnp.float32)
  h = jnp.maximum(h + b1_ref[...][None, :], 0.0)
  y2 = jnp.dot(h, w2_ref[...], preferred_element_type=jnp.float32)
  out_ref[...] = y2 * dinv_ref[...]


_mid = pl.pallas_call(
    _mid_body,
    grid=(NPAD // BN,),
    in_specs=[
        pl.BlockSpec((NC, BN, DIN // 2), lambda i: (0, i, 0)),
        pl.BlockSpec((BN, 1), lambda i: (i, 0)),
        pl.BlockSpec((DIN, DHID), lambda i: (0, 0)),
        pl.BlockSpec((DHID,), lambda i: (0,)),
        pl.BlockSpec((DHID, DOUT), lambda i: (0, 0)),
    ],
    out_specs=pl.BlockSpec((BN, DOUT), lambda i: (i, 0)),
    out_shape=jax.ShapeDtypeStruct((NPAD, DOUT), jnp.float32),
)


def _final_body(s2_ref, dinv_ref, b2_ref, out_ref):
  out_ref[...] = s2_ref[...] * dinv_ref[...] + b2_ref[...][None, :]


_final = pl.pallas_call(
    _final_body,
    grid=(NPAD // BN,),
    in_specs=[
        pl.BlockSpec((BN, DOUT), lambda i: (i, 0)),
        pl.BlockSpec((BN, 1), lambda i: (i, 0)),
        pl.BlockSpec((DOUT,), lambda i: (0,)),
    ],
    out_specs=pl.BlockSpec((BN, DOUT), lambda i: (i, 0)),
    out_shape=jax.ShapeDtypeStruct((N, DOUT), jnp.float32),
)


def kernel(x, edge_index, W1, b1, W2, b2):
  src3 = edge_index[0].reshape(NS, NB, K)
  dst3 = edge_index[1].reshape(NS, NB, K)
  ones = jnp.ones((NPAD,), jnp.float32)

  deg = _deg(dst3, ones)
  deg2 = deg.reshape(NPAD, 1)

  y1, dinv2 = _scale_split(x, deg2)
  s1 = _prop1(y1, src3, dst3)
  y2 = _mid(s1, dinv2, W1, b1, W2)
  s2 = _prop2(y2, src3, dst3)
  return _final(s2, dinv2, b2)
